# Initial kernel scaffold; baseline (speedup 1.0000x reference)
import functools

import jax
import jax.numpy as jnp
from jax import lax
from jax.experimental import pallas as pl
from jax.experimental.pallas import tpu as pltpu
from jax.experimental.pallas import tpu_sc as plsc

N_NODES = 100000
N_EDGES = 6400000
CUTOFF = 5.0

NW = 32
BLK = 128
CHUNK = 2048
BPC = CHUNK // BLK
NCHUNKS = N_EDGES // CHUNK
CH_BASE = NCHUNKS // NW
CH_EXTRA = NCHUNKS - CH_BASE * NW

_SW_COEF = (
    1.0,
    -2.4674003,
    2.02934625,
    -0.6675758,
    0.11751096,
    -0.012677814,
    0.0007968934,
)

_INV_C2 = 1.0 / (CUTOFF * CUTOFF)
_C2 = CUTOFF * CUTOFF


def _body(cp_hbm, srcb_hbm, dstb_hbm,
          vec_hbm, dist_hbm, sw_hbm, mask_hbm,
          shared, sidx, didx, srows, drows,
          vec_b, dist_b, sw_b, mask_b, sem, sem_s):
    cid = lax.axis_index("c")
    sid = lax.axis_index("s")
    wid = sid * 2 + cid

    @pl.when(sid == 0)
    def _():
        pltpu.async_copy(cp_hbm, shared, sem_s).wait()

    plsc.subcore_barrier()

    nch = CH_BASE + jnp.where(wid < CH_EXTRA, 1, 0)
    start = wid * CH_BASE + jnp.minimum(wid, CH_EXTRA)

    iota = lax.iota(jnp.int32, 16)
    col = [jnp.full((16,), c, jnp.int32) for c in range(3)]

    def chunk_body(ci, _):
        @pl.when(ci < nch)
        def _():
            chunk = start + ci
            base = chunk * CHUNK
            pltpu.sync_copy(srcb_hbm.at[pl.ds(chunk * BPC, BPC)], sidx)
            pltpu.sync_copy(dstb_hbm.at[pl.ds(chunk * BPC, BPC)], didx)
            cps = []
            for j in range(BPC):
                cps.append(pltpu.async_copy(shared.at[sidx.at[j]], srows.at[j], sem))
                cps.append(pltpu.async_copy(shared.at[didx.at[j]], drows.at[j], sem))
            for cp in cps:
                cp.wait()

            def blk_body(j, _):
                jv = jnp.full((16,), 1, jnp.int32) * j
                for h in range(BLK // 16):
                    pv = iota + (16 * h)
                    sx = plsc.load_gather(srows, [jv, pv, col[0]])
                    sy = plsc.load_gather(srows, [jv, pv, col[1]])
                    sz = plsc.load_gather(srows, [jv, pv, col[2]])
                    tx = plsc.load_gather(drows, [jv, pv, col[0]])
                    ty = plsc.load_gather(drows, [jv, pv, col[1]])
                    tz = plsc.load_gather(drows, [jv, pv, col[2]])
                    dx = tx - sx
                    dy = ty - sy
                    dz = tz - sz
                    d2 = dx * dx + dy * dy + dz * dz
                    d2g = jnp.maximum(d2, jnp.float32(1e-12))
                    ib = lax.bitcast_convert_type(d2g, jnp.int32)
                    ib = jnp.int32(0x5F3759DF) - lax.shift_right_arithmetic(ib, 1)
                    y = lax.bitcast_convert_type(ib, jnp.float32)
                    for _i in range(3):
                        y = y * (jnp.float32(1.5) - jnp.float32(0.5) * d2g * y * y)
                    dist = d2g * y
                    u = d2 * jnp.float32(_INV_C2)
                    acc = jnp.full((16,), _SW_COEF[-1], jnp.float32)
                    for c in _SW_COEF[-2::-1]:
                        acc = acc * u + jnp.float32(c)
                    lt = d2 < jnp.float32(_C2)
                    sw = jnp.where(lt, acc, jnp.float32(0.0))
                    m = jnp.where(lt, jnp.int32(1), jnp.int32(0))

                    pos = j * BLK + 16 * h
                    posv = jv * BLK + pv
                    vidx = posv * 3
                    plsc.store_scatter(vec_b, [vidx], dx)
                    plsc.store_scatter(vec_b, [vidx + 1], dy)
                    plsc.store_scatter(vec_b, [vidx + 2], dz)
                    dist_b[pl.ds(pos, 16)] = dist
                    sw_b[pl.ds(pos, 16)] = sw
                    mask_b[pl.ds(pos, 16)] = m

            lax.fori_loop(0, BPC, blk_body, None, unroll=1)

            pltpu.sync_copy(vec_b, vec_hbm.at[pl.ds(base * 3, CHUNK * 3)])
            pltpu.sync_copy(dist_b, dist_hbm.at[pl.ds(base, CHUNK)])
            pltpu.sync_copy(sw_b, sw_hbm.at[pl.ds(base, CHUNK)])
            pltpu.sync_copy(mask_b, mask_hbm.at[pl.ds(base, CHUNK)])

    lax.fori_loop(0, CH_BASE + 1, chunk_body, None, unroll=1)


@jax.jit
def kernel(coordinates, edge_src, edge_dst):
    cp = jnp.concatenate(
        [coordinates, jnp.zeros((N_NODES, 1), jnp.float32)], axis=1)
    srcb = edge_src.reshape(N_EDGES // BLK, BLK)
    dstb = edge_dst.reshape(N_EDGES // BLK, BLK)

    mesh = plsc.VectorSubcoreMesh(core_axis_name="c", subcore_axis_name="s")
    vec_f, dist, sw, mask_i = pl.kernel(
        _body,
        out_type=[
            jax.ShapeDtypeStruct((N_EDGES * 3,), jnp.float32),
            jax.ShapeDtypeStruct((N_EDGES,), jnp.float32),
            jax.ShapeDtypeStruct((N_EDGES,), jnp.float32),
            jax.ShapeDtypeStruct((N_EDGES,), jnp.int32),
        ],
        mesh=mesh,
        scratch_types=[
            pltpu.VMEM_SHARED((N_NODES, 4), jnp.float32),
            pltpu.VMEM((BPC, BLK), jnp.int32),
            pltpu.VMEM((BPC, BLK), jnp.int32),
            pltpu.VMEM((BPC, BLK, 4), jnp.float32),
            pltpu.VMEM((BPC, BLK, 4), jnp.float32),
            pltpu.VMEM((CHUNK * 3,), jnp.float32),
            pltpu.VMEM((CHUNK,), jnp.float32),
            pltpu.VMEM((CHUNK,), jnp.float32),
            pltpu.VMEM((CHUNK,), jnp.int32),
            pltpu.SemaphoreType.DMA,
            pltpu.SemaphoreType.DMA,
        ],
    )(cp, srcb, dstb)

    vec = vec_f.reshape(N_EDGES, 3)
    edge_mask = mask_i.astype(jnp.bool_)
    return (vec, dist, sw, edge_mask)


# speedup vs baseline: 10.8179x; 10.8179x over previous
import jax
import jax.numpy as jnp
from jax import lax
from jax.experimental import pallas as pl
from jax.experimental.pallas import tpu as pltpu
from jax.experimental.pallas import tpu_sc as plsc

N_NODES = 100000
N_EDGES = 6400000
CUTOFF = 5.0

NW = 32
BLK = 128
CHUNK = 2048
BPC = CHUNK // BLK
NCHUNKS = N_EDGES // CHUNK
CH_BASE = NCHUNKS // NW
CH_EXTRA = NCHUNKS - CH_BASE * NW

_SW_COEF = (
    1.0,
    -2.4674003,
    2.02934625,
    -0.6675758,
    0.11751096,
    -0.012677814,
    0.0007968934,
)

_INV_C2 = 1.0 / (CUTOFF * CUTOFF)
_C2 = CUTOFF * CUTOFF


def _body(cx_hbm, cy_hbm, cz_hbm, srcb_hbm, dstb_hbm,
          vec_hbm, dist_hbm, sw_hbm, mask_hbm,
          shx, shy, shz,
          sidx, didx, gsx, gsy, gsz, gdx, gdy, gdz,
          vec_b, dist_b, sw_b, mask_b, sem, sem_s):
    cid = lax.axis_index("c")
    sid = lax.axis_index("s")
    wid = sid * 2 + cid

    @pl.when(sid == 0)
    def _():
        cx = pltpu.async_copy(cx_hbm, shx, sem_s)
        cy = pltpu.async_copy(cy_hbm, shy, sem_s)
        cz = pltpu.async_copy(cz_hbm, shz, sem_s)
        cx.wait()
        cy.wait()
        cz.wait()

    plsc.subcore_barrier()

    nch = CH_BASE + jnp.where(wid < CH_EXTRA, 1, 0)
    start = wid * CH_BASE + jnp.minimum(wid, CH_EXTRA)

    iota = lax.iota(jnp.int32, 16)

    def chunk_body(ci, _):
        @pl.when(ci < nch)
        def _():
            chunk = start + ci
            base = chunk * CHUNK
            pltpu.sync_copy(srcb_hbm.at[pl.ds(chunk * BPC, BPC)], sidx)
            pltpu.sync_copy(dstb_hbm.at[pl.ds(chunk * BPC, BPC)], didx)
            cps = []
            for j in range(BPC):
                d = pl.ds(j * BLK, BLK)
                si = sidx.at[j]
                di = didx.at[j]
                cps.append(pltpu.async_copy(shx.at[si], gsx.at[d], sem))
                cps.append(pltpu.async_copy(shy.at[si], gsy.at[d], sem))
                cps.append(pltpu.async_copy(shz.at[si], gsz.at[d], sem))
                cps.append(pltpu.async_copy(shx.at[di], gdx.at[d], sem))
                cps.append(pltpu.async_copy(shy.at[di], gdy.at[d], sem))
                cps.append(pltpu.async_copy(shz.at[di], gdz.at[d], sem))
            for cp in cps:
                cp.wait()

            def grp_body(g, _):
                pos = g * 16
                posv = iota + pos
                s16 = pl.ds(pos, 16)
                dx = gdx[s16] - gsx[s16]
                dy = gdy[s16] - gsy[s16]
                dz = gdz[s16] - gsz[s16]
                d2 = dx * dx + dy * dy + dz * dz
                d2g = jnp.maximum(d2, jnp.float32(1e-12))
                ib = lax.bitcast_convert_type(d2g, jnp.int32)
                ib = jnp.int32(0x5F3759DF) - lax.shift_right_arithmetic(ib, 1)
                y = lax.bitcast_convert_type(ib, jnp.float32)
                for _i in range(3):
                    y = y * (jnp.float32(1.5) - jnp.float32(0.5) * d2g * y * y)
                dist = d2g * y
                u = d2 * jnp.float32(_INV_C2)
                acc = jnp.full((16,), _SW_COEF[-1], jnp.float32)
                for c in _SW_COEF[-2::-1]:
                    acc = acc * u + jnp.float32(c)
                lt = d2 < jnp.float32(_C2)
                sw = jnp.where(lt, acc, jnp.float32(0.0))
                m = jnp.where(lt, jnp.int32(1), jnp.int32(0))

                vidx = posv * 3
                plsc.store_scatter(vec_b, [vidx], dx)
                plsc.store_scatter(vec_b, [vidx + 1], dy)
                plsc.store_scatter(vec_b, [vidx + 2], dz)
                dist_b[s16] = dist
                sw_b[s16] = sw
                mask_b[s16] = m

            lax.fori_loop(0, CHUNK // 16, grp_body, None, unroll=4)

            pltpu.sync_copy(vec_b, vec_hbm.at[pl.ds(base * 3, CHUNK * 3)])
            pltpu.sync_copy(dist_b, dist_hbm.at[pl.ds(base, CHUNK)])
            pltpu.sync_copy(sw_b, sw_hbm.at[pl.ds(base, CHUNK)])
            pltpu.sync_copy(mask_b, mask_hbm.at[pl.ds(base, CHUNK)])

    lax.fori_loop(0, CH_BASE + 1, chunk_body, None, unroll=1)


@jax.jit
def kernel(coordinates, edge_src, edge_dst):
    cx0 = coordinates[:, 0]
    cy0 = coordinates[:, 1]
    cz0 = coordinates[:, 2]
    srcb = edge_src.reshape(N_EDGES // BLK, BLK)
    dstb = edge_dst.reshape(N_EDGES // BLK, BLK)

    mesh = plsc.VectorSubcoreMesh(core_axis_name="c", subcore_axis_name="s")
    vec_f, dist, sw, mask_i = pl.kernel(
        _body,
        out_type=[
            jax.ShapeDtypeStruct((N_EDGES * 3,), jnp.float32),
            jax.ShapeDtypeStruct((N_EDGES,), jnp.float32),
            jax.ShapeDtypeStruct((N_EDGES,), jnp.float32),
            jax.ShapeDtypeStruct((N_EDGES,), jnp.int32),
        ],
        mesh=mesh,
        compiler_params=pltpu.CompilerParams(needs_layout_passes=False),
        scratch_types=[
            pltpu.VMEM_SHARED((N_NODES,), jnp.float32),
            pltpu.VMEM_SHARED((N_NODES,), jnp.float32),
            pltpu.VMEM_SHARED((N_NODES,), jnp.float32),
            pltpu.VMEM((BPC, BLK), jnp.int32),
            pltpu.VMEM((BPC, BLK), jnp.int32),
            pltpu.VMEM((CHUNK,), jnp.float32),
            pltpu.VMEM((CHUNK,), jnp.float32),
            pltpu.VMEM((CHUNK,), jnp.float32),
            pltpu.VMEM((CHUNK,), jnp.float32),
            pltpu.VMEM((CHUNK,), jnp.float32),
            pltpu.VMEM((CHUNK,), jnp.float32),
            pltpu.VMEM((CHUNK * 3,), jnp.float32),
            pltpu.VMEM((CHUNK,), jnp.float32),
            pltpu.VMEM((CHUNK,), jnp.float32),
            pltpu.VMEM((CHUNK,), jnp.int32),
            pltpu.SemaphoreType.DMA,
            pltpu.SemaphoreType.DMA,
        ],
    )(cx0, cy0, cz0, srcb, dstb)

    vec = vec_f.reshape(N_EDGES, 3)
    edge_mask = mask_i.astype(jnp.bool_)
    return (vec, dist, sw, edge_mask)
